# baseline (speedup 1.0000x reference)
import functools

import numpy as np
import jax
from jax._src.config import enable_x64 as _enable_x64
import jax.numpy as jnp
from jax import lax
from jax.experimental import pallas as pl
from jax.experimental.pallas import tpu as pltpu
from jax.experimental.pallas import tpu_sc as plsc

B = 4096
S = 200
D = 32

NC = 2
NS = 16
NW = NC * NS
RPW = B // NW
CA = 104
CB = 96
RPG = 4
NGRP = RPW // RPG


def _i32(x):
  return np.int32(x) if isinstance(x, int) else x


def _sc_embed_sum(words32, table):
  mesh = plsc.VectorSubcoreMesh(
      core_axis_name="c", subcore_axis_name="s", num_cores=NC,
      num_subcores=NS)

  @functools.partial(
      pl.kernel,
      out_type=jax.ShapeDtypeStruct((B, D), jnp.float32),
      mesh=mesh,
      scratch_types=[
          pltpu.VMEM((RPW, S), jnp.int32),
          pltpu.VMEM((2, RPG, CA, D), jnp.float32),
          pltpu.VMEM((2, RPG, CB, D), jnp.float32),
          pltpu.VMEM((RPW, D), jnp.float32),
          pltpu.SemaphoreType.DMA,
          pltpu.SemaphoreType.DMA,
      ],
      compiler_params=pltpu.CompilerParams(use_tc_tiling_on_sc=False),
  )
  def sc_kernel(words_hbm, table_hbm, out_hbm, idx_v, bufs_a, bufs_b, out_v,
                sem_a, sem_b):
    wid = lax.axis_index("s") * NC + lax.axis_index("c")
    pltpu.sync_copy(words_hbm.at[pl.ds(wid * RPW, RPW)], idx_v)

    sems = (sem_a, sem_b)

    def row_refs(g, i):
      row = _i32(g * RPG + i)
      return (table_hbm.at[idx_v.at[row, pl.ds(np.int32(0), CA)]],
              table_hbm.at[idx_v.at[row, pl.ds(np.int32(CA), CB)]])

    def issue(g, slot):
      for i in range(RPG):
        ra, rb = row_refs(g, i)
        pltpu.async_copy(ra, bufs_a.at[np.int32(slot), np.int32(i)],
                         sems[slot])
        pltpu.async_copy(rb, bufs_b.at[np.int32(slot), np.int32(i)],
                         sems[slot])

    def drain(g, slot):
      for i in range(RPG):
        ra, rb = row_refs(g, i)
        pltpu.make_async_copy(ra, bufs_a.at[np.int32(slot), np.int32(i)],
                              sems[slot]).wait()
        pltpu.make_async_copy(rb, bufs_b.at[np.int32(slot), np.int32(i)],
                              sems[slot]).wait()

    def process(g, slot):
      for i in range(RPG):
        z = jnp.zeros((16,), jnp.float32)
        sl = np.int32(slot)
        ii = np.int32(i)

        def acc_body(r, carry):
          a00, a01, a10, a11 = carry
          a00 = a00 + bufs_a[sl, ii, r, 0:16]
          a01 = a01 + bufs_a[sl, ii, r, 16:32]
          a10 = a10 + bufs_b[sl, ii, r, 0:16]
          a11 = a11 + bufs_b[sl, ii, r, 16:32]
          return a00, a01, a10, a11

        def tail_body(r, carry):
          a00, a01 = carry
          a00 = a00 + bufs_a[sl, ii, r, 0:16]
          a01 = a01 + bufs_a[sl, ii, r, 16:32]
          return a00, a01

        a00, a01, a10, a11 = lax.fori_loop(
            np.int32(0), np.int32(CB), acc_body, (z, z, z, z), unroll=12)
        a00, a01 = lax.fori_loop(
            np.int32(CB), np.int32(CA), tail_body, (a00, a01), unroll=8)
        row = _i32(g * RPG + i)
        out_v[row, 0:16] = a00 + a10
        out_v[row, 16:32] = a01 + a11

    issue(0, 0)

    def group_body(t, carry):
      g = t * np.int32(2)
      issue(g + 1, 1)
      drain(g, 0)
      process(g, 0)

      @pl.when(g + 2 < NGRP)
      def _():
        issue(g + 2, 0)

      drain(g + 1, 1)
      process(g + 1, 1)
      return carry

    lax.fori_loop(np.int32(0), np.int32(NGRP // 2), group_body, np.int32(0))

    pltpu.sync_copy(out_v, out_hbm.at[pl.ds(wid * RPW, RPW)])

  return sc_kernel(words32, table)


_BB = 512


def _pos_bits_kernel(pos_ref, out_ref):
  p = pos_ref[...]
  p = jnp.concatenate(
      [p, jnp.zeros((_BB, 256 - S), jnp.int32)], axis=1)

  planes = [p]
  w = 256
  while w > 1:
    h = w // 2
    nxt = []
    carry = None
    for pj in planes:
      lo = pj[:, :h]
      hi = pj[:, h:]
      if carry is None:
        s = lo ^ hi
        c = lo & hi
      else:
        x = lo ^ hi
        s = x ^ carry
        c = (lo & hi) | (carry & x)
      nxt.append(s)
      carry = c
    nxt.append(carry)
    planes = nxt
    w = h

  d_iota = lax.broadcasted_iota(jnp.int32, (1, D), 1)
  acc = jnp.zeros((_BB, D), jnp.int32)
  for j, pj in enumerate(planes):
    acc = acc + (((pj >> d_iota) & 1) << j)
  out_ref[...] = acc.astype(jnp.float32)


def _tc_pos_sum(pos32):
  return pl.pallas_call(
      _pos_bits_kernel,
      out_shape=jax.ShapeDtypeStruct((B, D), jnp.float32),
      grid=(B // _BB,),
      in_specs=[pl.BlockSpec((_BB, S), lambda i: (i, 0))],
      out_specs=pl.BlockSpec((_BB, D), lambda i: (i, 0)),
  )(pos32)


@jax.jit
def kernel(words, positions, table):
  with _enable_x64(False):
    words32 = words.astype(jnp.int32)
    pos32 = positions.astype(jnp.int32)
    emb = _sc_embed_sum(words32, table)
    pos_sum = _tc_pos_sum(pos32)
    return emb + pos_sum


# speedup vs baseline: 1.2874x; 1.2874x over previous
import functools

import numpy as np
import jax
from jax._src.config import enable_x64 as _enable_x64
import jax.numpy as jnp
from jax import lax
from jax.experimental import pallas as pl
from jax.experimental.pallas import tpu as pltpu
from jax.experimental.pallas import tpu_sc as plsc

B = 4096
S = 200
D = 32

NC = 2
NS = 16
NW = NC * NS
RPW = B // NW
CA = 128
CB = 72
RPG = 4
NGRP = RPW // RPG


def _i32(x):
  return np.int32(x) if isinstance(x, int) else x


def _sc_embed_sum(words32, table):
  mesh = plsc.VectorSubcoreMesh(
      core_axis_name="c", subcore_axis_name="s", num_cores=NC,
      num_subcores=NS)

  @functools.partial(
      pl.kernel,
      out_type=jax.ShapeDtypeStruct((B, D), jnp.float32),
      mesh=mesh,
      scratch_types=[
          pltpu.VMEM((2 * RPW, 128), jnp.int32),
          pltpu.VMEM((2, RPG, CA, D), jnp.float32),
          pltpu.VMEM((2, RPG, CB, D), jnp.float32),
          pltpu.VMEM((RPW, D), jnp.float32),
          pltpu.SemaphoreType.DMA,
          pltpu.SemaphoreType.DMA,
      ],
      compiler_params=pltpu.CompilerParams(use_tc_tiling_on_sc=False),
  )
  def sc_kernel(words_hbm, table_hbm, out_hbm, idx_v, bufs_a, bufs_b, out_v,
                sem_a, sem_b):
    wid = lax.axis_index("s") * NC + lax.axis_index("c")
    pltpu.sync_copy(words_hbm.at[pl.ds(wid * (2 * RPW), 2 * RPW)], idx_v)

    sems = (sem_a, sem_b)

    def row_refs(g, i):
      r = g * RPG + i
      a_row = (r // 8) * 16 + r % 8
      return (table_hbm.at[idx_v.at[_i32(a_row)]],
              table_hbm.at[idx_v.at[_i32(a_row + 8), pl.ds(np.int32(0), CB)]])

    def issue(g, slot):
      for i in range(RPG):
        ra, rb = row_refs(g, i)
        pltpu.async_copy(ra, bufs_a.at[np.int32(slot), np.int32(i)],
                         sems[slot])
        pltpu.async_copy(rb, bufs_b.at[np.int32(slot), np.int32(i)],
                         sems[slot])

    def drain(g, slot):
      for i in range(RPG):
        ra, rb = row_refs(g, i)
        pltpu.make_async_copy(ra, bufs_a.at[np.int32(slot), np.int32(i)],
                              sems[slot]).wait()
        pltpu.make_async_copy(rb, bufs_b.at[np.int32(slot), np.int32(i)],
                              sems[slot]).wait()

    def process(g, slot):
      for i in range(RPG):
        z = jnp.zeros((16,), jnp.float32)
        sl = np.int32(slot)
        ii = np.int32(i)

        def acc_body(r, carry):
          a00, a01, a10, a11 = carry
          a00 = a00 + bufs_a[sl, ii, r, 0:16]
          a01 = a01 + bufs_a[sl, ii, r, 16:32]
          a10 = a10 + bufs_b[sl, ii, r, 0:16]
          a11 = a11 + bufs_b[sl, ii, r, 16:32]
          return a00, a01, a10, a11

        def tail_body(r, carry):
          a00, a01 = carry
          a00 = a00 + bufs_a[sl, ii, r, 0:16]
          a01 = a01 + bufs_a[sl, ii, r, 16:32]
          return a00, a01

        a00, a01, a10, a11 = lax.fori_loop(
            np.int32(0), np.int32(CB), acc_body, (z, z, z, z), unroll=12)
        a00, a01 = lax.fori_loop(
            np.int32(CB), np.int32(CA), tail_body, (a00, a01), unroll=8)
        row = _i32(g * RPG + i)
        out_v[row, 0:16] = a00 + a10
        out_v[row, 16:32] = a01 + a11

    issue(0, 0)

    def group_body(t, carry):
      g = t * np.int32(2)
      issue(g + 1, 1)
      drain(g, 0)
      process(g, 0)

      @pl.when(g + 2 < NGRP)
      def _():
        issue(g + 2, 0)

      drain(g + 1, 1)
      process(g + 1, 1)
      return carry

    lax.fori_loop(np.int32(0), np.int32(NGRP // 2), group_body, np.int32(0))

    pltpu.sync_copy(out_v, out_hbm.at[pl.ds(wid * RPW, RPW)])

  return sc_kernel(words32, table)


_BB = 512


def _pos_bits_kernel(pos_ref, out_ref):
  p = pos_ref[...]
  p = jnp.concatenate(
      [p, jnp.zeros((_BB, 256 - S), jnp.int32)], axis=1)

  planes = [p]
  w = 256
  while w > 1:
    h = w // 2
    nxt = []
    carry = None
    for pj in planes:
      lo = pj[:, :h]
      hi = pj[:, h:]
      if carry is None:
        s = lo ^ hi
        c = lo & hi
      else:
        x = lo ^ hi
        s = x ^ carry
        c = (lo & hi) | (carry & x)
      nxt.append(s)
      carry = c
    nxt.append(carry)
    planes = nxt
    w = h

  d_iota = lax.broadcasted_iota(jnp.int32, (1, D), 1)
  acc = jnp.zeros((_BB, D), jnp.int32)
  for j, pj in enumerate(planes):
    acc = acc + (((pj >> d_iota) & 1) << j)
  out_ref[...] = acc.astype(jnp.float32)


def _tc_pos_sum(pos32):
  return pl.pallas_call(
      _pos_bits_kernel,
      out_shape=jax.ShapeDtypeStruct((B, D), jnp.float32),
      grid=(B // _BB,),
      in_specs=[pl.BlockSpec((_BB, S), lambda i: (i, 0))],
      out_specs=pl.BlockSpec((_BB, D), lambda i: (i, 0)),
  )(pos32)


@jax.jit
def kernel(words, positions, table):
  with _enable_x64(False):
    words32 = words.astype(jnp.int32)
    pos32 = positions.astype(jnp.int32)
    words_x = jnp.pad(words32, ((0, 0), (0, 56)))
    words_x = words_x.reshape(B // 8, 8, 2, 128).transpose(0, 2, 1, 3)
    words_x = words_x.reshape(2 * B, 128)
    emb = _sc_embed_sum(words_x, table)
    pos_sum = _tc_pos_sum(pos32)
    return emb + pos_sum
